# Initial kernel scaffold; baseline (speedup 1.0000x reference)
import jax
import jax.numpy as jnp
from jax.experimental import pallas as pl


def kernel(x, edge_index, W, b, conv_time):
    raise NotImplementedError("write your pallas kernel here")


# speedup vs baseline: 16.8822x; 16.8822x over previous
import functools

import jax
import jax.numpy as jnp
from jax import lax
from jax.experimental import pallas as pl
from jax.experimental.pallas import tpu as pltpu
from jax.experimental.pallas import tpu_sc as plsc

NC = 2
NS = 16
EB = 128
ECHUNK = 8
CW = 16


def _mesh():
    return plsc.VectorSubcoreMesh(
        core_axis_name="c", subcore_axis_name="s", num_cores=NC,
        num_subcores=NS)


def _make_deg_kernel(NP, EPW):
    rows_per_tile = NP // NS
    chunks = EPW // (NC * NS * ECHUNK)

    @functools.partial(
        pl.kernel,
        out_type=jax.ShapeDtypeStruct((NC * NP,), jnp.float32),
        mesh=_mesh(),
        compiler_params=pltpu.CompilerParams(use_tc_tiling_on_sc=False),
        scratch_types=[
            pltpu.VMEM((ECHUNK, EB), jnp.int32),
            pltpu.VMEM((EB,), jnp.float32),
            pltpu.VMEM((rows_per_tile,), jnp.float32),
            pltpu.VMEM_SHARED((NP,), jnp.float32),
        ],
    )
    def deg_kernel(dst2, zeros_np, ones_eb, degp, dstb, ones_v, zbuf, deg_sh):
        c = lax.axis_index("c")
        s = lax.axis_index("s")
        r0 = s * rows_per_tile
        pltpu.sync_copy(zeros_np.at[pl.ds(r0, rows_per_tile)], zbuf)
        pltpu.sync_copy(zbuf, deg_sh.at[pl.ds(r0, rows_per_tile)])
        pltpu.sync_copy(ones_eb, ones_v)
        plsc.subcore_barrier()

        wid = c * NS + s
        row_base = wid * chunks * ECHUNK

        def body(j, carry):
            pltpu.sync_copy(dst2.at[pl.ds(row_base + j * ECHUNK, ECHUNK)],
                            dstb)
            for k in range(ECHUNK):
                pltpu.sync_copy(ones_v, deg_sh.at[dstb.at[k]], add=True)
            return carry

        lax.fori_loop(0, chunks, body, 0)
        plsc.subcore_barrier()
        pltpu.sync_copy(deg_sh.at[pl.ds(r0, rows_per_tile)], zbuf)
        pltpu.sync_copy(zbuf, degp.at[pl.ds(c * NP + r0, rows_per_tile)])

    return deg_kernel


def _make_agg_kernel(NP, NPASS, EPW):
    rows_per_tile = NP // NS
    rchunks = rows_per_tile // EB
    erows_per_tile = EPW // (NC * NS)
    chunks = erows_per_tile // ECHUNK

    @functools.partial(
        pl.kernel,
        out_type=jax.ShapeDtypeStruct((NC * NPASS * NP, CW), jnp.float32),
        mesh=_mesh(),
        compiler_params=pltpu.CompilerParams(use_tc_tiling_on_sc=False),
        scratch_types=[
            pltpu.VMEM((ECHUNK, EB), jnp.int32),
            pltpu.VMEM((ECHUNK, EB), jnp.int32),
            pltpu.VMEM((EB, CW), jnp.float32),
            pltpu.VMEM((EB, CW), jnp.float32),
            pltpu.VMEM_SHARED((NP, CW), jnp.float32),
            pltpu.SemaphoreType.DMA,
        ],
    )
    def agg_kernel(g3f, srcg, dst2, zeros_row, out, srcb, dstb, rows_v,
                   zrow_v, acc_sh, gsem):
        c = lax.axis_index("c")
        s = lax.axis_index("s")
        r0 = s * rows_per_tile
        wid = c * NS + s
        erow_base = wid * erows_per_tile
        pltpu.sync_copy(zeros_row, zrow_v)

        for p in range(NPASS):
            for q in range(rchunks):
                pltpu.sync_copy(zrow_v, acc_sh.at[pl.ds(r0 + q * EB, EB)])
            plsc.subcore_barrier()

            def body(j, carry):
                rb = erow_base + j * ECHUNK
                pltpu.sync_copy(srcg.at[pl.ds(p * EPW + rb, ECHUNK)], srcb)
                pltpu.sync_copy(dst2.at[pl.ds(rb, ECHUNK)], dstb)
                for k in range(ECHUNK):
                    pltpu.async_copy(
                        g3f.at[srcb.at[k]], rows_v, gsem).wait()
                    pltpu.sync_copy(rows_v, acc_sh.at[dstb.at[k]], add=True)
                return carry

            lax.fori_loop(0, chunks, body, 0)
            plsc.subcore_barrier()
            plane = (c * NPASS + p) * NP
            for q in range(rchunks):
                pltpu.sync_copy(acc_sh.at[pl.ds(r0 + q * EB, EB)], rows_v)
                pltpu.sync_copy(rows_v,
                                out.at[pl.ds(plane + r0 + q * EB, EB)])
            plsc.subcore_barrier()

    return agg_kernel


def _lin_kernel(x_ref, w_ref, degp_ref, g3_ref, dinv_ref, *, C, NPASS):
    h = jnp.dot(x_ref[...], w_ref[...], preferred_element_type=jnp.float32)
    deg = degp_ref[0] + degp_ref[1]
    dinv = lax.rsqrt(deg)
    g = h * dinv[:, None]
    for p in range(NPASS):
        lo = p * CW
        if lo + CW <= C:
            g3_ref[p] = g[:, lo:lo + CW]
        else:
            pad = lo + CW - C
            g3_ref[p] = jnp.concatenate(
                [g[:, lo:C], jnp.zeros((g.shape[0], pad), jnp.float32)],
                axis=1)
    dinv_ref[...] = dinv


def _out_kernel(acc_ref, dinv_ref, b_ref, o_ref, *, C, NPASS):
    a = acc_ref[...]
    dinv = dinv_ref[...]
    full = jnp.concatenate(
        [a[p] + a[NPASS + p] for p in range(NPASS)], axis=1)
    o_ref[...] = full[:, :C] * dinv[:, None] + b_ref[...][None, :]


def _rescale_kernel(acc_ref, dinv_ref, g3_ref, *, NPASS):
    a = acc_ref[...]
    d2 = dinv_ref[...] * dinv_ref[...]
    for p in range(NPASS):
        g3_ref[p] = (a[p] + a[NPASS + p]) * d2[:, None]


def kernel(x, edge_index, W, b, conv_time):
    N, D = x.shape
    C = W.shape[1]
    E = edge_index.shape[1]
    NPASS = pl.cdiv(C, CW)

    NP = pl.cdiv(N, NS * EB) * NS * EB
    egrain = NC * NS * ECHUNK * EB
    EP = pl.cdiv(E + N, egrain) * egrain
    EPW = EP // EB

    src = edge_index[0]
    dst = edge_index[1]
    padn = EP - E - N
    loops = jnp.arange(N, dtype=edge_index.dtype)
    ar = jnp.arange(padn, dtype=edge_index.dtype)
    src1 = jnp.concatenate([src, loops, ar % N]).reshape(1, EPW, EB)
    dst2 = jnp.concatenate(
        [dst, loops, N + (ar % (NP - N))]).reshape(EPW, EB)
    offs = (jnp.arange(NPASS, dtype=edge_index.dtype) * NP)[:, None, None]
    srcg = (src1 + offs).reshape(NPASS * EPW, EB)

    zeros_np = jnp.zeros((NP,), jnp.float32)
    ones_eb = jnp.ones((EB,), jnp.float32)
    zeros_row = jnp.zeros((EB, CW), jnp.float32)

    degp = _make_deg_kernel(NP, EPW)(dst2, zeros_np, ones_eb)
    degp = degp.reshape(NC, NP)

    BN = 1024
    nblk = pl.cdiv(N, BN)
    g3, dinv = pl.pallas_call(
        functools.partial(_lin_kernel, C=C, NPASS=NPASS),
        grid=(nblk,),
        in_specs=[
            pl.BlockSpec((BN, D), lambda i: (i, 0)),
            pl.BlockSpec((D, C), lambda i: (0, 0)),
            pl.BlockSpec((NC, BN), lambda i: (0, i)),
        ],
        out_specs=[
            pl.BlockSpec((NPASS, BN, CW), lambda i: (0, i, 0)),
            pl.BlockSpec((BN,), lambda i: (i,)),
        ],
        out_shape=[
            jax.ShapeDtypeStruct((NPASS, NP, CW), jnp.float32),
            jax.ShapeDtypeStruct((NP,), jnp.float32),
        ],
    )(x, W, degp)

    agg = _make_agg_kernel(NP, NPASS, EPW)

    def step(g3a):
        acc = agg(g3a.reshape(NPASS * NP, CW), srcg, dst2, zeros_row)
        return acc.reshape(NC * NPASS, NP, CW)

    def body(_, g3a):
        acc = step(g3a)
        return pl.pallas_call(
            functools.partial(_rescale_kernel, NPASS=NPASS),
            grid=(nblk,),
            in_specs=[
                pl.BlockSpec((NC * NPASS, BN, CW), lambda i: (0, i, 0)),
                pl.BlockSpec((BN,), lambda i: (i,)),
            ],
            out_specs=pl.BlockSpec((NPASS, BN, CW), lambda i: (0, i, 0)),
            out_shape=jax.ShapeDtypeStruct((NPASS, NP, CW), jnp.float32),
        )(acc, dinv)

    g3 = lax.fori_loop(0, conv_time - 1, body, g3)
    acc = step(g3)

    out = pl.pallas_call(
        functools.partial(_out_kernel, C=C, NPASS=NPASS),
        grid=(nblk,),
        in_specs=[
            pl.BlockSpec((NC * NPASS, BN, CW), lambda i: (0, i, 0)),
            pl.BlockSpec((BN,), lambda i: (i,)),
            pl.BlockSpec((C,), lambda i: (0,)),
        ],
        out_specs=pl.BlockSpec((BN, C), lambda i: (i, 0)),
        out_shape=jax.ShapeDtypeStruct((N, C), jnp.float32),
    )(acc, dinv, b)
    return out
